# Initial kernel scaffold; baseline (speedup 1.0000x reference)
import jax
import jax.numpy as jnp
from jax.experimental import pallas as pl


def kernel(batch_k_head_softmax):
    raise NotImplementedError("write your pallas kernel here")


# speedup vs baseline: 1.5675x; 1.5675x over previous
import jax
import jax.numpy as jnp
from jax import lax
from jax.experimental import pallas as pl
from jax.experimental.pallas import tpu as pltpu

_B, _K, _N = 128, 16, 32768
_ROWS = _B * _K
_BR = 64
_NBLK = _ROWS // _BR


def _argmax_body(x_ref, o_ref):
    x = x_ref[...]
    m = jnp.max(x, axis=1, keepdims=True)
    idx = lax.broadcasted_iota(jnp.int32, x.shape, 1)
    masked = jnp.where(x == m, idx, _N)
    o_ref[0, 0, :] = jnp.min(masked, axis=1)


def kernel(batch_k_head_softmax):
    x = batch_k_head_softmax.reshape(_ROWS, _N)
    out = pl.pallas_call(
        _argmax_body,
        grid=(_NBLK,),
        in_specs=[pl.BlockSpec((_BR, _N), lambda i: (i, 0))],
        out_specs=pl.BlockSpec((1, 1, _BR), lambda i: (i, 0, 0)),
        out_shape=jax.ShapeDtypeStruct((_NBLK, 1, _BR), jnp.int32),
        compiler_params=pltpu.CompilerParams(
            dimension_semantics=("arbitrary",),
        ),
    )(x)
    return out.reshape(_B, _K)
